# baseline (speedup 1.0000x reference)
import jax
import jax.numpy as jnp
from jax import lax
from jax.experimental import pallas as pl
from jax.experimental.pallas import tpu as pltpu
from jax.experimental.pallas import tpu_sc as plsc

N = 10000
E = 320000
HID = 128
NUM_RELS = 16

NC = 2
NS = 16
NW = NC * NS

N_PAD = 10240
ROWS_PER_TILE = N_PAD // NS
CHUNK = 128
NBUF = 2
NCHUNK = 80
EPW = NCHUNK * CHUNK
E_PAD = NW * EPW
KMAX = NCHUNK // NBUF

_EROWS = E // HID
_EROWS_PAD = E_PAD // HID
EDW = 4 * CHUNK


def _dense_body(x_ref, w_ref, att_ref, rel_ref, wrel_ref, conf_ref,
                ei_ref, et_ref,
                h_ref, s_ref, c_ref, ed_ref):
    x = x_ref[...]
    h = lax.dot_general(x, w_ref[...], (((1,), (1,)), ((), ())),
                        preferred_element_type=jnp.float32)
    h_ref[...] = h
    att = att_ref[...]
    s_ref[...] = lax.dot_general(att[0:2, :], h, (((1,), (1,)), ((), ())),
                                 preferred_element_type=jnp.float32)
    rproj = lax.dot_general(rel_ref[...], wrel_ref[...],
                            (((1,), (1,)), ((), ())),
                            preferred_element_type=jnp.float32)
    c_ref[...] = lax.dot_general(att[2:3, :], rproj, (((1,), (1,)), ((), ())),
                                 preferred_element_type=jnp.float32)
    npad_rows = _EROWS_PAD - _EROWS
    spread = (lax.broadcasted_iota(jnp.int32, (npad_rows, HID), 0) * HID
              + lax.broadcasted_iota(jnp.int32, (npad_rows, HID), 1)) % N
    qbits = lax.bitcast_convert_type(
        jnp.sqrt(jnp.maximum(conf_ref[...], 1e-6)), jnp.int32)
    ed_ref[0:_EROWS, 0:HID] = ei_ref[0]
    ed_ref[_EROWS:_EROWS_PAD, 0:HID] = spread
    ed_ref[0:_EROWS, HID:2 * HID] = ei_ref[1]
    ed_ref[_EROWS:_EROWS_PAD, HID:2 * HID] = spread
    ed_ref[0:_EROWS, 2 * HID:3 * HID] = et_ref[...]
    ed_ref[_EROWS:_EROWS_PAD, 2 * HID:3 * HID] = jnp.zeros(
        (npad_rows, HID), jnp.int32)
    ed_ref[0:_EROWS, 3 * HID:4 * HID] = qbits
    ed_ref[_EROWS:_EROWS_PAD, 3 * HID:4 * HID] = jnp.zeros(
        (npad_rows, HID), jnp.int32)


_dense_call = pl.pallas_call(
    _dense_body,
    out_shape=[
        jax.ShapeDtypeStruct((N, HID), jnp.float32),
        jax.ShapeDtypeStruct((2, N), jnp.float32),
        jax.ShapeDtypeStruct((1, NUM_RELS), jnp.float32),
        jax.ShapeDtypeStruct((_EROWS_PAD, EDW), jnp.int32),
    ],
)


_mesh = plsc.VectorSubcoreMesh(core_axis_name="c", subcore_axis_name="s")

_sc_scratch = (
    [pltpu.VMEM_SHARED((N_PAD, HID), jnp.float32),
     pltpu.VMEM_SHARED((N_PAD,), jnp.float32),
     pltpu.VMEM_SHARED((N,), jnp.float32),
     pltpu.VMEM_SHARED((N,), jnp.float32),
     pltpu.VMEM((NUM_RELS,), jnp.float32)]
    + [pltpu.VMEM((EDW,), jnp.int32)] * NBUF
    + [pltpu.VMEM((CHUNK,), jnp.int32)] * NBUF
    + [pltpu.VMEM((CHUNK,), jnp.int32)] * NBUF
    + [pltpu.VMEM((CHUNK,), jnp.float32)] * NBUF
    + [pltpu.VMEM((CHUNK,), jnp.float32)] * NBUF
    + [pltpu.VMEM((CHUNK,), jnp.float32)] * NBUF
    + [pltpu.VMEM((CHUNK, HID), jnp.float32)] * NBUF
    + [pltpu.SemaphoreType.DMA] * (4 * NBUF)
)


def _sc_body(h_hbm, s_hbm, c_hbm, ed_hbm,
             acc_out, den_out, *scr):
    acc_sh, den_sh, s1_sh, s2_sh, c_v = scr[0:5]
    o = 5
    ed_b = scr[o:o + NBUF]; o += NBUF
    src_b = scr[o:o + NBUF]; o += NBUF
    ds_b = scr[o:o + NBUF]; o += NBUF
    w_b = scr[o:o + NBUF]; o += NBUF
    s1_b = scr[o:o + NBUF]; o += NBUF
    s2_b = scr[o:o + NBUF]; o += NBUF
    h_b = scr[o:o + NBUF]; o += NBUF
    semi = scr[o:o + NBUF]; o += NBUF
    semg = scr[o:o + NBUF]; o += NBUF
    semt = scr[o:o + NBUF]; o += NBUF
    sems = scr[o:o + NBUF]; o += NBUF

    cid = lax.axis_index("c")
    sid = lax.axis_index("s")
    zeros16 = jnp.zeros((16,), jnp.float32)

    wid = cid * NS + sid
    cbase = wid * NCHUNK

    def idx_issue(p, ch):
        pltpu.async_copy(ed_hbm.at[cbase + ch], ed_b[p], semi[p])

    def idx_drain(p):
        pltpu.make_async_copy(ed_hbm.at[0], ed_b[p], semi[p]).wait()

    def extract(p):
        for i in range(CHUNK // 16):
            src_b[p][pl.ds(i * 16, 16)] = ed_b[p][pl.ds(i * 16, 16)]
            ds_b[p][pl.ds(i * 16, 16)] = ed_b[p][pl.ds(CHUNK + i * 16, 16)]

    def gather_issue(p):
        pltpu.async_copy(h_hbm.at[src_b[p]], h_b[p], semg[p])

    def gather_wait(p):
        pltpu.make_async_copy(h_hbm.at[src_b[p]], h_b[p], semg[p]).wait()

    def sgather_issue(p):
        pltpu.async_copy(s1_sh.at[ds_b[p]], s1_b[p], semt[p])
        pltpu.async_copy(s2_sh.at[src_b[p]], s2_b[p], semt[p])

    def sgather_drain(p):
        pltpu.make_async_copy(s1_sh.at[ds_b[p]], s1_b[p], semt[p]).wait()
        pltpu.make_async_copy(s2_sh.at[src_b[p]], s2_b[p], semt[p]).wait()

    def scat_issue(p):
        pltpu.async_copy(h_b[p], acc_sh.at[ds_b[p]], sems[p], add=True)
        pltpu.async_copy(w_b[p], den_sh.at[ds_b[p]], sems[p], add=True)

    def scat_drain(p):
        pltpu.make_async_copy(h_b[p], acc_sh.at[ds_b[p]], sems[p]).wait()
        pltpu.make_async_copy(w_b[p], den_sh.at[ds_b[p]], sems[p]).wait()

    def compute_w(p):
        for i in range(CHUNK // 16):
            sl = pl.ds(i * 16, 16)
            t16 = jnp.clip(ed_b[p][pl.ds(2 * CHUNK + i * 16, 16)],
                           0, NUM_RELS - 1)
            q16 = plsc.bitcast(ed_b[p][pl.ds(3 * CHUNK + i * 16, 16)],
                               jnp.float32)
            cc = plsc.load_gather(c_v, [t16])
            e = s1_b[p][sl] + s2_b[p][sl] + cc
            e = jnp.maximum(e, 0.2 * e)
            w_b[p][sl] = jnp.exp(e) * q16

    def scale(p):
        def _sgroup(g, c2):
            wg = w_b[p][pl.ds(g * 16, 16)]
            for j in range(16):
                sw = wg[j]
                row = g * 16 + j
                for k in range(HID // 16):
                    slk = pl.ds(k * 16, 16)
                    h_b[p][row, slk] = h_b[p][row, slk] * sw
            return c2
        lax.fori_loop(0, CHUNK // 16, _sgroup, 0)

    for p in range(NBUF):
        idx_issue(p, p)

    def _zrow(j, carry):
        for k in range(HID // 16):
            h_b[0][j, pl.ds(k * 16, 16)] = zeros16
        return carry
    lax.fori_loop(0, CHUNK, _zrow, 0)
    for i in range(CHUNK // 16):
        w_b[0][pl.ds(i * 16, 16)] = zeros16
    for b in range(ROWS_PER_TILE // CHUNK):
        rs = sid * ROWS_PER_TILE + b * CHUNK
        pltpu.sync_copy(h_b[0], acc_sh.at[pl.ds(rs, CHUNK), :])
        pltpu.sync_copy(w_b[0], den_sh.at[pl.ds(rs, CHUNK)])

    @pl.when(sid == 0)
    def _():
        pltpu.sync_copy(s_hbm.at[0], s1_sh)
        pltpu.sync_copy(s_hbm.at[1], s2_sh)
    pltpu.sync_copy(c_hbm, c_v)

    idx_drain(0)
    extract(0)
    gather_issue(0)
    plsc.subcore_barrier()
    sgather_issue(0)

    def body(k, carry):
        for j in range(NBUF):
            ch = k * NBUF + j
            p = j
            p1 = (j + 1) % NBUF
            sgather_drain(p)
            compute_w(p)
            if j == NBUF - 1:
                scat_drain(p1)

                @pl.when(k < KMAX - 1)
                def _():
                    idx_drain(p1)
                    extract(p1)
                    gather_issue(p1)
                    sgather_issue(p1)
            else:
                @pl.when(k > 0)
                def _():
                    scat_drain(p1)
                idx_drain(p1)
                extract(p1)
                gather_issue(p1)
                sgather_issue(p1)
            gather_wait(p)
            scale(p)
            scat_issue(p)

            @pl.when(k < KMAX - 1)
            def _():
                idx_issue(p, ch + NBUF)
        return carry

    lax.fori_loop(0, KMAX, body, 0)

    scat_drain(1)

    plsc.subcore_barrier()

    for b in range(ROWS_PER_TILE // CHUNK):
        rs = sid * ROWS_PER_TILE + b * CHUNK
        pltpu.sync_copy(acc_sh.at[pl.ds(rs, CHUNK), :],
                        acc_out.at[cid, pl.ds(rs, CHUNK), :])
        pltpu.sync_copy(den_sh.at[pl.ds(rs, CHUNK)],
                        den_out.at[cid, pl.ds(rs, CHUNK)])


_sc_call = pl.kernel(
    _sc_body,
    out_type=[
        jax.ShapeDtypeStruct((NC, N_PAD, HID), jnp.float32),
        jax.ShapeDtypeStruct((NC, N_PAD), jnp.float32),
    ],
    mesh=_mesh,
    compiler_params=pltpu.CompilerParams(needs_layout_passes=False),
    scratch_types=_sc_scratch,
)


def _combine_body(acc_ref, den_ref, bias_ref, out_ref):
    a = acc_ref[0, 0:N, :] + acc_ref[1, 0:N, :]
    d = den_ref[0, 0:N, :] + den_ref[1, 0:N, :]
    out_ref[...] = a / (d + 1e-16) + bias_ref[...]


_combine_call = pl.pallas_call(
    _combine_body,
    out_shape=jax.ShapeDtypeStruct((N, HID), jnp.float32),
)


def kernel(x, edge_index, edge_type_in, edge_attr, W_msg, rel_emb, W_rel,
           att_vec, bias):
    att3 = att_vec.reshape(3, HID)
    conf2d = edge_attr.reshape(_EROWS, HID)
    ei2d = edge_index.reshape(2, _EROWS, HID)
    et2d = edge_type_in.reshape(_EROWS, HID)
    h, s, crel, ed = _dense_call(
        x, W_msg, att3, rel_emb, W_rel, conf2d, ei2d, et2d)

    acc, den = _sc_call(h, s, crel.reshape(NUM_RELS), ed)
    return _combine_call(acc, den[:, :, None], bias.reshape(1, HID))
